# baseline (speedup 1.0000x reference)
import functools

import jax
import jax.numpy as jnp
from jax.experimental import pallas as pl
from jax.experimental.pallas import tpu as pltpu

D = 128
ROLE_SUBJ = 1
ROLE_OBJ = 2
ROLE_VERB = 3
ROLE_CONJ = 5


def _dense_body(x_ref, rid_ref, W_role_ref, b_role_ref, W_spawn_ref,
                W_subj_ref, b_subj_ref, W_obj_ref, b_obj_ref, W_vg_ref, b_vg_ref,
                W_vc_ref, b_vc_ref, W_rot_ref, b_rot_ref, W_og_ref, b_og_ref,
                ln_g_ref, ln_b_ref,
                role_ref, spawn_ref, A_ref, bcoef_ref, xout_ref):
    x = x_ref[...]
    r = rid_ref[...]

    role_ref[...] = jnp.dot(x, W_role_ref[...],
                            preferred_element_type=jnp.float32) + b_role_ref[...]
    spawn_ref[...] = jnp.sum(x * W_spawn_ref[...], axis=-1, keepdims=True)

    vec_subj = jnp.dot(x, W_subj_ref[...], preferred_element_type=jnp.float32) + b_subj_ref[...]
    vec_obj = jnp.dot(x, W_obj_ref[...], preferred_element_type=jnp.float32) + b_obj_ref[...]
    vgate = jax.nn.sigmoid(
        jnp.sum(x * W_vg_ref[...], axis=-1, keepdims=True) + b_vg_ref[...])
    vec_vc = jnp.dot(x, W_vc_ref[...], preferred_element_type=jnp.float32) + b_vc_ref[...]
    vec_rot = jnp.dot(x, W_rot_ref[...], preferred_element_type=jnp.float32) + b_rot_ref[...]

    A = jnp.where(r == ROLE_SUBJ, vec_subj,
        jnp.where(r == ROLE_OBJ, vec_obj,
        jnp.where(r == ROLE_VERB, vgate * vec_vc,
        jnp.where(r == ROLE_CONJ, x, vec_rot))))
    A_ref[...] = A
    bcoef = jnp.where(r == ROLE_VERB, 1.0 - vgate, 0.0)
    bcoef_ref[...] = jnp.broadcast_to(bcoef, bcoef_ref.shape)

    gate_out = jax.nn.sigmoid(
        jnp.dot(x, W_og_ref[...], preferred_element_type=jnp.float32) + b_og_ref[...])
    h = x + gate_out * x
    mu = jnp.mean(h, axis=-1, keepdims=True)
    var = jnp.mean((h - mu) ** 2, axis=-1, keepdims=True)
    xout_ref[...] = (h - mu) * jax.lax.rsqrt(var + 1e-5) * ln_g_ref[...] + ln_b_ref[...]


@functools.partial(jax.jit, static_argnames=("blk",))
def _dense(x, role_ids, W_role, b_role, W_spawn, W_subj, b_subj, W_obj, b_obj,
           W_vg, b_vg, W_vc, b_vc, W_rot, b_rot, W_og, b_og, ln_g, ln_b, blk=512):
    B = x.shape[0]
    grid = (B // blk,)
    row = lambda i: (i, 0)
    rep = lambda i: (0, 0)
    out_shapes = (
        jax.ShapeDtypeStruct((B, 9), jnp.float32),
        jax.ShapeDtypeStruct((B, 1), jnp.float32),
        jax.ShapeDtypeStruct((B, D), jnp.float32),
        jax.ShapeDtypeStruct((B, 16), jnp.float32),
        jax.ShapeDtypeStruct((B, D), jnp.float32),
    )
    in_specs = [
        pl.BlockSpec((blk, D), row),
        pl.BlockSpec((blk, 1), row),
        pl.BlockSpec((D, 9), rep),
        pl.BlockSpec((1, 9), rep),
        pl.BlockSpec((1, D), rep),
        pl.BlockSpec((D, D), rep),
        pl.BlockSpec((1, D), rep),
        pl.BlockSpec((D, D), rep),
        pl.BlockSpec((1, D), rep),
        pl.BlockSpec((1, D), rep),
        pl.BlockSpec((1, 1), rep),
        pl.BlockSpec((D, D), rep),
        pl.BlockSpec((1, D), rep),
        pl.BlockSpec((D, D), rep),
        pl.BlockSpec((1, D), rep),
        pl.BlockSpec((D, D), rep),
        pl.BlockSpec((1, D), rep),
        pl.BlockSpec((1, D), rep),
        pl.BlockSpec((1, D), rep),
    ]
    out_specs = (
        pl.BlockSpec((blk, 9), row),
        pl.BlockSpec((blk, 1), row),
        pl.BlockSpec((blk, D), row),
        pl.BlockSpec((blk, 16), row),
        pl.BlockSpec((blk, D), row),
    )
    return pl.pallas_call(
        _dense_body,
        grid=grid,
        in_specs=in_specs,
        out_specs=out_specs,
        out_shape=out_shapes,
    )(x, role_ids, W_role, b_role, W_spawn, W_subj, b_subj, W_obj, b_obj,
      W_vg, b_vg, W_vc, b_vc, W_rot, b_rot, W_og, b_og, ln_g, ln_b)


def kernel(x, memory, role_ids, slot_idx, W_role, b_role, W_spawn, b_spawn,
           W_subj, b_subj, W_obj, b_obj, W_vg, b_vg, W_vc, b_vc, W_rot, b_rot,
           W_og, b_og, ln_g, ln_b):
    B = x.shape[0]
    M = memory.shape[0]
    role_logits, spawn, A, bcoef, x_out = _dense(
        x, role_ids.astype(jnp.int32).reshape(B, 1),
        W_role, (b_role + 0.0).reshape(1, 9), W_spawn.reshape(1, D),
        W_subj, b_subj.reshape(1, D), W_obj, b_obj.reshape(1, D),
        W_vg.reshape(1, D), b_vg.reshape(1, 1), W_vc, b_vc.reshape(1, D),
        W_rot, b_rot.reshape(1, D), W_og, b_og.reshape(1, D),
        ln_g.reshape(1, D), ln_b.reshape(1, D))
    spawn_logit = spawn.reshape(B) + b_spawn[0]
    order = jnp.arange(B, dtype=jnp.int32)
    winner = jnp.full((M,), -1, jnp.int32).at[slot_idx].max(order)
    is_winner = winner[slot_idx] == order
    old = memory[slot_idx]
    write_val = A + bcoef[:, :1] * old
    tgt = jnp.where(is_winner, slot_idx, M)
    memory_out = memory.at[tgt].set(write_val, mode="drop")
    return (x_out, role_logits, spawn_logit, memory_out)
